# baseline (speedup 1.0000x reference)
import functools

import jax
import jax.numpy as jnp
from jax import lax
from jax.experimental import pallas as pl
from jax.experimental.pallas import tpu as pltpu
from jax.experimental.pallas import tpu_sc as plsc

E = 160000
T = 160000
D = 64
DP = 128
NCORES = 2
NSUB = 16
NW = NCORES * NSUB


_BE = 2000


def _pre_body(x_ref, rbf_ref, wrbf_ref, wji_ref, bji_ref, wkj_ref, bkj_ref,
              xji_ref, xk_ref):
    xv = x_ref[...]
    g = jnp.dot(rbf_ref[...], wrbf_ref[...], preferred_element_type=jnp.float32)
    xji_ref[...] = jnp.dot(xv, wji_ref[...],
                           preferred_element_type=jnp.float32) + bji_ref[...]
    xk = (jnp.dot(xv, wkj_ref[...], preferred_element_type=jnp.float32)
          + bkj_ref[...]) * g
    xk_ref[...] = jnp.concatenate([xk, jnp.zeros_like(xk)], axis=1)


def _pre_call(x, rbf, W_rbf, W_ji, b_ji, W_kj, b_kj):
    n = x.shape[0] // _BE
    nr = rbf.shape[1]
    return pl.pallas_call(
        _pre_body,
        grid=(n,),
        in_specs=[
            pl.BlockSpec((_BE, D), lambda i: (i, 0)),
            pl.BlockSpec((_BE, nr), lambda i: (i, 0)),
            pl.BlockSpec((nr, D), lambda i: (0, 0)),
            pl.BlockSpec((D, D), lambda i: (0, 0)),
            pl.BlockSpec((1, D), lambda i: (0, 0)),
            pl.BlockSpec((D, D), lambda i: (0, 0)),
            pl.BlockSpec((1, D), lambda i: (0, 0)),
        ],
        out_specs=[
            pl.BlockSpec((_BE, D), lambda i: (i, 0)),
            pl.BlockSpec((_BE, DP), lambda i: (i, 0)),
        ],
        out_shape=[
            jax.ShapeDtypeStruct((x.shape[0], D), jnp.float32),
            jax.ShapeDtypeStruct((x.shape[0], DP), jnp.float32),
        ],
    )(x, rbf, W_rbf, W_ji, b_ji.reshape(1, D), W_kj, b_kj.reshape(1, D))


_BT = 1280


def _bil_body(sbf_ref, xg_ref, wsbf_ref, wb_ref, out_ref):
    bt = sbf_ref.shape[0]
    sbf_e = jnp.dot(sbf_ref[...], wsbf_ref[...],
                    preferred_element_type=jnp.float32)
    sbf_et = sbf_e.astype(jnp.bfloat16).T
    xgt = xg_ref[...][:, :D].astype(jnp.bfloat16).T
    mt = (sbf_et[:, None, :] * xgt[None, :, :]).reshape(D * D, bt)
    res = jnp.dot(wb_ref[...], mt, preferred_element_type=jnp.float32).T
    out_ref[...] = jnp.concatenate([res, jnp.zeros_like(res)], axis=1)


def _bil_call(sbf, xg, W_sbf, Wb):
    n = sbf.shape[0] // _BT
    ns = sbf.shape[1]
    return pl.pallas_call(
        _bil_body,
        grid=(n,),
        in_specs=[
            pl.BlockSpec((_BT, ns), lambda i: (i, 0)),
            pl.BlockSpec((_BT, DP), lambda i: (i, 0)),
            pl.BlockSpec((ns, D), lambda i: (0, 0)),
            pl.BlockSpec((D, D * D), lambda i: (0, 0)),
        ],
        out_specs=pl.BlockSpec((_BT, DP), lambda i: (i, 0)),
        out_shape=jax.ShapeDtypeStruct((sbf.shape[0], DP), jnp.float32),
    )(sbf, xg, W_sbf, Wb.astype(jnp.bfloat16))


def _silu(v):
    return v * jax.nn.sigmoid(v)


def _post_body(x_ref, xji_ref, seg_ref,
               wb1_ref, bb1_ref, wb2_ref, bb2_ref, wfbs_ref, bfbs_ref,
               wa11_ref, ba11_ref, wa12_ref, ba12_ref,
               wa21_ref, ba21_ref, wa22_ref, ba22_ref, out_ref):
    def dot(a, w_ref, b_ref):
        return jnp.dot(a.astype(jnp.bfloat16), w_ref[...],
                       preferred_element_type=jnp.float32) + b_ref[...]

    h = xji_ref[...] + seg_ref[...][:, :D]
    h = h + dot(_silu(dot(h, wb1_ref, bb1_ref)), wb2_ref, bb2_ref)
    h = _silu(h)
    h = _silu(dot(h, wfbs_ref, bfbs_ref))
    xo = x_ref[...] + h
    xo = xo + dot(_silu(dot(xo, wa11_ref, ba11_ref)), wa12_ref, ba12_ref)
    xo = _silu(xo)
    xo = xo + dot(_silu(dot(xo, wa21_ref, ba21_ref)), wa22_ref, ba22_ref)
    out_ref[...] = _silu(xo)


def _post_call(x, x_ji, seg, Wb1, bb1, Wb2, bb2, W_fbs, b_fbs,
               Wa11, ba11, Wa12, ba12, Wa21, ba21, Wa22, ba22):
    n = x.shape[0] // _BE
    mat = pl.BlockSpec((D, D), lambda i: (0, 0))
    vec = pl.BlockSpec((1, D), lambda i: (0, 0))
    big = pl.BlockSpec((_BE, D), lambda i: (i, 0))
    segspec = pl.BlockSpec((_BE, DP), lambda i: (i, 0))
    return pl.pallas_call(
        _post_body,
        grid=(n,),
        in_specs=[big, big, segspec,
                  mat, vec, mat, vec, mat, vec,
                  mat, vec, mat, vec, mat, vec, mat, vec],
        out_specs=big,
        out_shape=jax.ShapeDtypeStruct((x.shape[0], D), jnp.float32),
    )(x, x_ji, seg,
      Wb1.astype(jnp.bfloat16), bb1.reshape(1, D),
      Wb2.astype(jnp.bfloat16), bb2.reshape(1, D),
      W_fbs.astype(jnp.bfloat16), b_fbs.reshape(1, D),
      Wa11.astype(jnp.bfloat16), ba11.reshape(1, D),
      Wa12.astype(jnp.bfloat16), ba12.reshape(1, D),
      Wa21.astype(jnp.bfloat16), ba21.reshape(1, D),
      Wa22.astype(jnp.bfloat16), ba22.reshape(1, D))


_GQ = 256
_GROWS = 640
_GPW = _GROWS // NW


def _gather_body(xk_hbm, id3_hbm, out_hbm, idx_v, rows_v, isem, gsem, osem):
    w = lax.axis_index("s") * NCORES + lax.axis_index("c")
    nslot = 3

    store_started, store_waited = [], []

    def wait_store(j):
        pltpu.make_async_copy(rows_v.at[j % nslot],
                              out_hbm.at[w * _GPW + j], osem).wait()
        store_waited.append(j)

    for k in range(_GPW + 2):
        if k >= 2:
            km2 = k - 2
            pltpu.make_async_copy(
                xk_hbm.at[idx_v.at[km2 % nslot, 0]], rows_v.at[km2 % nslot],
                gsem).wait()
            pltpu.async_copy(rows_v.at[km2 % nslot],
                             out_hbm.at[w * _GPW + km2], osem)
            store_started.append(km2)
        if k < _GPW:
            pltpu.async_copy(id3_hbm.at[w * _GPW + k], idx_v.at[k % nslot],
                             isem)
        if 1 <= k <= _GPW:
            km1 = k - 1
            if km1 - nslot >= 0 and (km1 - nslot) not in store_waited:
                wait_store(km1 - nslot)
            pltpu.make_async_copy(id3_hbm.at[w * _GPW + km1],
                                  idx_v.at[km1 % nslot], isem).wait()
            pltpu.async_copy(
                xk_hbm.at[idx_v.at[km1 % nslot, 0]], rows_v.at[km1 % nslot],
                gsem)
    for j in store_started:
        if j not in store_waited:
            wait_store(j)


def _gather_call(xk, id2):
    mesh = plsc.VectorSubcoreMesh(core_axis_name="c", subcore_axis_name="s")
    f = pl.kernel(
        _gather_body,
        out_type=jax.ShapeDtypeStruct((_GROWS, _GQ, DP), jnp.float32),
        mesh=mesh,
        scratch_types=[
            pltpu.VMEM((3, 1, _GQ), jnp.int32),
            pltpu.VMEM((3, _GQ, DP), jnp.float32),
            pltpu.SemaphoreType.DMA,
            pltpu.SemaphoreType.DMA,
            pltpu.SemaphoreType.DMA,
        ],
    )
    return f(xk, id2)


_NCHUNK = 16
_CH = E // _NCHUNK
_CHP = 10240
_IDROWS_PW = 80
_FQ = 128
_SQ = 256


def _scatter_body(bil_hbm, ids_hbm, zeros_hbm, seg_hbm,
                  acc_sh, ids_v, widx1, didx1, widx2, didx2, rows_v,
                  ssem, asem):
    c = lax.axis_index("c")
    s = lax.axis_index("s")
    iota = lax.iota(jnp.int32, 16)

    pltpu.sync_copy(ids_hbm.at[pl.ds(s * _IDROWS_PW, _IDROWS_PW)], ids_v)
    wbase0 = s * (_IDROWS_PW * _FQ)

    def wait_add():
        pltpu.make_async_copy(bil_hbm.at[pl.ds(0, _SQ)], rows_v, asem).wait()

    def flush(qidx):
        lax.cond(qidx > 0, wait_add, lambda: None)
        for j in range(_SQ // 16):
            widx2[0, pl.ds(j * 16, 16)] = widx1[pl.ds(j * 16, 16)]
            didx2[0, pl.ds(j * 16, 16)] = didx1[pl.ds(j * 16, 16)]
        pltpu.async_copy(bil_hbm.at[widx2.at[0]], rows_v, ssem).wait()
        pltpu.async_copy(rows_v, acc_sh.at[didx2.at[0]], asem, add=True)

    for cc in range(_NCHUNK // NCORES):
        chunk = c * (_NCHUNK // NCORES) + cc
        lo = chunk * _CH
        zrows = _CHP // NSUB
        pltpu.sync_copy(zeros_hbm, acc_sh.at[pl.ds(s * zrows, zrows)])
        plsc.subcore_barrier()

        def step(r, carry):
            cnt, qidx = carry
            data = []
            for j in range(8):
                ids16 = ids_v[r, 0, pl.ds(j * 16, 16)]
                local = ids16 - lo
                m = local.astype(jnp.uint32) < jnp.uint32(_CH)
                csum = jnp.cumsum(m.astype(jnp.int32))
                data.append((local, m, csum, jnp.max(csum), j))
            run = cnt
            for local, m, csum, tot, j in data:
                pos = run - 1 + csum
                plsc.store_scatter(didx1, [pos], local, mask=m)
                w16 = wbase0 + r * _FQ + j * 16 + iota
                plsc.store_scatter(widx1, [pos], w16, mask=m)
                run = run + tot
            cnt = run

            def do_flush(carry2):
                cn, qi = carry2
                flush(qi)
                for j2 in range(8):
                    wtail = widx1[pl.ds(_SQ + j2 * 16, 16)]
                    dtail = didx1[pl.ds(_SQ + j2 * 16, 16)]
                    widx1[pl.ds(j2 * 16, 16)] = wtail
                    didx1[pl.ds(j2 * 16, 16)] = dtail
                return cn - _SQ, qi + 1

            return lax.cond(cnt >= _SQ, do_flush, lambda x: x, (cnt, qidx))

        cnt, qidx = lax.fori_loop(0, _IDROWS_PW, step,
                                  (jnp.int32(0), jnp.int32(0)))

        for j in range(_SQ // 16):
            pos = j * 16 + iota
            keep = pos < cnt
            dv = didx1[pl.ds(j * 16, 16)]
            wv = widx1[pl.ds(j * 16, 16)]
            didx1[pl.ds(j * 16, 16)] = jnp.where(keep, dv, _CH + iota)
            widx1[pl.ds(j * 16, 16)] = jnp.where(keep, wv, iota * 64)
        flush(qidx)
        wait_add()

        plsc.subcore_barrier()
        pltpu.sync_copy(acc_sh.at[pl.ds(s * 624, 624)],
                        seg_hbm.at[pl.ds(lo + s * 624, 624)])

        @pl.when(s == NSUB - 1)
        def _():
            pltpu.sync_copy(acc_sh.at[pl.ds(9360 + 624, 16)],
                            seg_hbm.at[pl.ds(lo + 9360 + 624, 16)])
        plsc.subcore_barrier()


def _scatter_call(bil, ids, zeros):
    mesh = plsc.VectorSubcoreMesh(core_axis_name="c", subcore_axis_name="s")
    f = pl.kernel(
        _scatter_body,
        out_type=jax.ShapeDtypeStruct((E, DP), jnp.float32),
        compiler_params=pltpu.CompilerParams(needs_layout_passes=False),
        mesh=mesh,
        scratch_types=[
            pltpu.VMEM_SHARED((_CHP, DP), jnp.float32),
            pltpu.VMEM((_IDROWS_PW, 1, _FQ), jnp.int32),
            pltpu.VMEM((2 * _SQ,), jnp.int32),
            pltpu.VMEM((2 * _SQ,), jnp.int32),
            pltpu.VMEM((1, _SQ), jnp.int32),
            pltpu.VMEM((1, _SQ), jnp.int32),
            pltpu.VMEM((_SQ, DP), jnp.float32),
            pltpu.SemaphoreType.DMA,
            pltpu.SemaphoreType.DMA,
        ],
    )
    return f(bil, ids, zeros)


def kernel(x, rbf, sbf, id_expand_kj, id_reduce_ji,
           W_rbf, W_sbf, W_ji, b_ji, W_kj, b_kj, W_bilin,
           Wb1, bb1, Wb2, bb2, W_fbs, b_fbs,
           Wa11, ba11, Wa12, ba12, Wa21, ba21, Wa22, ba22):
    x_ji, xk = _pre_call(x, rbf, W_rbf, W_ji, b_ji, W_kj, b_kj)

    pad = _GROWS * _GQ - T
    idp = jnp.concatenate(
        [id_expand_kj, (jnp.arange(pad, dtype=jnp.int32) * 37) % E])
    id2 = idp.reshape(_GROWS, 1, _GQ)
    xg3 = _gather_call(xk, id2)
    xg = xg3.reshape(_GROWS * _GQ, DP)

    Wb = W_bilin.reshape(D, D * D)
    bil = _bil_call(sbf, xg, W_sbf, Wb)

    idr = jnp.concatenate(
        [id_reduce_ji, jnp.full((pad,), E, jnp.int32)]).reshape(
            NSUB * _IDROWS_PW, 1, _FQ)
    zeros = jnp.zeros((_CHP // NSUB, DP), jnp.float32)
    seg = _scatter_call(bil, idr, zeros)

    return _post_call(x, x_ji, seg, Wb1, bb1, Wb2, bb2, W_fbs, b_fbs,
                      Wa11, ba11, Wa12, ba12, Wa21, ba21, Wa22, ba22)


# speedup vs baseline: 1.0547x; 1.0547x over previous
import functools

import jax
import jax.numpy as jnp
from jax import lax
from jax.experimental import pallas as pl
from jax.experimental.pallas import tpu as pltpu
from jax.experimental.pallas import tpu_sc as plsc

E = 160000
T = 160000
D = 64
DP = 128
NCORES = 2
NSUB = 16
NW = NCORES * NSUB


_BE = 2000


def _pre_body(x_ref, rbf_ref, wrbf_ref, wji_ref, bji_ref, wkj_ref, bkj_ref,
              xji_ref, xk_ref):
    xv = x_ref[...]
    g = jnp.dot(rbf_ref[...], wrbf_ref[...], preferred_element_type=jnp.float32)
    xji_ref[...] = jnp.dot(xv, wji_ref[...],
                           preferred_element_type=jnp.float32) + bji_ref[...]
    xk = (jnp.dot(xv, wkj_ref[...], preferred_element_type=jnp.float32)
          + bkj_ref[...]) * g
    xk_ref[...] = jnp.concatenate([xk, jnp.zeros_like(xk)], axis=1)


def _pre_call(x, rbf, W_rbf, W_ji, b_ji, W_kj, b_kj):
    n = x.shape[0] // _BE
    nr = rbf.shape[1]
    return pl.pallas_call(
        _pre_body,
        grid=(n,),
        in_specs=[
            pl.BlockSpec((_BE, D), lambda i: (i, 0)),
            pl.BlockSpec((_BE, nr), lambda i: (i, 0)),
            pl.BlockSpec((nr, D), lambda i: (0, 0)),
            pl.BlockSpec((D, D), lambda i: (0, 0)),
            pl.BlockSpec((1, D), lambda i: (0, 0)),
            pl.BlockSpec((D, D), lambda i: (0, 0)),
            pl.BlockSpec((1, D), lambda i: (0, 0)),
        ],
        out_specs=[
            pl.BlockSpec((_BE, D), lambda i: (i, 0)),
            pl.BlockSpec((_BE, DP), lambda i: (i, 0)),
        ],
        out_shape=[
            jax.ShapeDtypeStruct((x.shape[0], D), jnp.float32),
            jax.ShapeDtypeStruct((x.shape[0], DP), jnp.float32),
        ],
    )(x, rbf, W_rbf, W_ji, b_ji.reshape(1, D), W_kj, b_kj.reshape(1, D))


_BT = 1280


def _bil_body(sbf_ref, xg_ref, wsbf_ref, wb_ref, out_ref):
    bt = sbf_ref.shape[0]
    sbf_e = jnp.dot(sbf_ref[...], wsbf_ref[...],
                    preferred_element_type=jnp.float32)
    sbf_et = sbf_e.astype(jnp.bfloat16).T
    xgt = xg_ref[...][:, :D].astype(jnp.bfloat16).T
    mt = (sbf_et[:, None, :] * xgt[None, :, :]).reshape(D * D, bt)
    res = jnp.dot(wb_ref[...], mt, preferred_element_type=jnp.float32).T
    out_ref[...] = jnp.concatenate([res, jnp.zeros_like(res)], axis=1)


def _bil_call(sbf, xg, W_sbf, Wb):
    n = sbf.shape[0] // _BT
    ns = sbf.shape[1]
    return pl.pallas_call(
        _bil_body,
        grid=(n,),
        in_specs=[
            pl.BlockSpec((_BT, ns), lambda i: (i, 0)),
            pl.BlockSpec((_BT, DP), lambda i: (i, 0)),
            pl.BlockSpec((ns, D), lambda i: (0, 0)),
            pl.BlockSpec((D, D * D), lambda i: (0, 0)),
        ],
        out_specs=pl.BlockSpec((_BT, DP), lambda i: (i, 0)),
        out_shape=jax.ShapeDtypeStruct((sbf.shape[0], DP), jnp.float32),
    )(sbf, xg, W_sbf, Wb.astype(jnp.bfloat16))


def _silu(v):
    return v * jax.nn.sigmoid(v)


def _post_body(x_ref, xji_ref, seg_ref,
               wb1_ref, bb1_ref, wb2_ref, bb2_ref, wfbs_ref, bfbs_ref,
               wa11_ref, ba11_ref, wa12_ref, ba12_ref,
               wa21_ref, ba21_ref, wa22_ref, ba22_ref, out_ref):
    def dot(a, w_ref, b_ref):
        return jnp.dot(a.astype(jnp.bfloat16), w_ref[...],
                       preferred_element_type=jnp.float32) + b_ref[...]

    h = xji_ref[...] + seg_ref[...][:, :D]
    h = h + dot(_silu(dot(h, wb1_ref, bb1_ref)), wb2_ref, bb2_ref)
    h = _silu(h)
    h = _silu(dot(h, wfbs_ref, bfbs_ref))
    xo = x_ref[...] + h
    xo = xo + dot(_silu(dot(xo, wa11_ref, ba11_ref)), wa12_ref, ba12_ref)
    xo = _silu(xo)
    xo = xo + dot(_silu(dot(xo, wa21_ref, ba21_ref)), wa22_ref, ba22_ref)
    out_ref[...] = _silu(xo)


def _post_call(x, x_ji, seg, Wb1, bb1, Wb2, bb2, W_fbs, b_fbs,
               Wa11, ba11, Wa12, ba12, Wa21, ba21, Wa22, ba22):
    n = x.shape[0] // _BE
    mat = pl.BlockSpec((D, D), lambda i: (0, 0))
    vec = pl.BlockSpec((1, D), lambda i: (0, 0))
    big = pl.BlockSpec((_BE, D), lambda i: (i, 0))
    segspec = pl.BlockSpec((_BE, DP), lambda i: (i, 0))
    return pl.pallas_call(
        _post_body,
        grid=(n,),
        in_specs=[big, big, segspec,
                  mat, vec, mat, vec, mat, vec,
                  mat, vec, mat, vec, mat, vec, mat, vec],
        out_specs=big,
        out_shape=jax.ShapeDtypeStruct((x.shape[0], D), jnp.float32),
    )(x, x_ji, seg,
      Wb1.astype(jnp.bfloat16), bb1.reshape(1, D),
      Wb2.astype(jnp.bfloat16), bb2.reshape(1, D),
      W_fbs.astype(jnp.bfloat16), b_fbs.reshape(1, D),
      Wa11.astype(jnp.bfloat16), ba11.reshape(1, D),
      Wa12.astype(jnp.bfloat16), ba12.reshape(1, D),
      Wa21.astype(jnp.bfloat16), ba21.reshape(1, D),
      Wa22.astype(jnp.bfloat16), ba22.reshape(1, D))


_GQ = 128
_GROWS = 1280
_GPW = _GROWS // NW


def _gather_body(xk_hbm, id3_hbm, out_hbm, idx_v, rows_v, isem, gsem, osem):
    w = lax.axis_index("s") * NCORES + lax.axis_index("c")
    nslot = 3

    store_started, store_waited = [], []

    def wait_store(j):
        pltpu.make_async_copy(rows_v.at[j % nslot],
                              out_hbm.at[w * _GPW + j], osem).wait()
        store_waited.append(j)

    for k in range(_GPW + 2):
        if k >= 2:
            km2 = k - 2
            pltpu.make_async_copy(
                xk_hbm.at[idx_v.at[km2 % nslot, 0]], rows_v.at[km2 % nslot],
                gsem).wait()
            pltpu.async_copy(rows_v.at[km2 % nslot],
                             out_hbm.at[w * _GPW + km2], osem)
            store_started.append(km2)
        if k < _GPW:
            pltpu.async_copy(id3_hbm.at[w * _GPW + k], idx_v.at[k % nslot],
                             isem)
        if 1 <= k <= _GPW:
            km1 = k - 1
            if km1 - nslot >= 0 and (km1 - nslot) not in store_waited:
                wait_store(km1 - nslot)
            pltpu.make_async_copy(id3_hbm.at[w * _GPW + km1],
                                  idx_v.at[km1 % nslot], isem).wait()
            pltpu.async_copy(
                xk_hbm.at[idx_v.at[km1 % nslot, 0]], rows_v.at[km1 % nslot],
                gsem)
    for j in store_started:
        if j not in store_waited:
            wait_store(j)


def _gather_call(xk, id2):
    mesh = plsc.VectorSubcoreMesh(core_axis_name="c", subcore_axis_name="s")
    f = pl.kernel(
        _gather_body,
        out_type=jax.ShapeDtypeStruct((_GROWS, _GQ, DP), jnp.float32),
        mesh=mesh,
        scratch_types=[
            pltpu.VMEM((3, 1, _GQ), jnp.int32),
            pltpu.VMEM((3, _GQ, DP), jnp.float32),
            pltpu.SemaphoreType.DMA,
            pltpu.SemaphoreType.DMA,
            pltpu.SemaphoreType.DMA,
        ],
    )
    return f(xk, id2)


_NCHUNK = 16
_CH = E // _NCHUNK
_CHP = 10240
_IDROWS_PW = 80
_FQ = 128
_SQ = 128


def _scatter_body(bil_hbm, ids_hbm, zeros_hbm, seg_hbm,
                  acc_sh, ids_v, widx1, didx1, widx2, didx2, rows_v,
                  ssem, asem):
    c = lax.axis_index("c")
    s = lax.axis_index("s")
    iota = lax.iota(jnp.int32, 16)

    pltpu.sync_copy(ids_hbm.at[pl.ds(s * _IDROWS_PW, _IDROWS_PW)], ids_v)
    wbase0 = s * (_IDROWS_PW * _FQ)

    def wait_add():
        pltpu.make_async_copy(bil_hbm.at[pl.ds(0, _SQ)], rows_v, asem).wait()

    def flush(qidx):
        lax.cond(qidx > 0, wait_add, lambda: None)
        for j in range(_SQ // 16):
            widx2[0, pl.ds(j * 16, 16)] = widx1[pl.ds(j * 16, 16)]
            didx2[0, pl.ds(j * 16, 16)] = didx1[pl.ds(j * 16, 16)]
        pltpu.async_copy(bil_hbm.at[widx2.at[0]], rows_v, ssem).wait()
        pltpu.async_copy(rows_v, acc_sh.at[didx2.at[0]], asem, add=True)

    for cc in range(_NCHUNK // NCORES):
        chunk = c * (_NCHUNK // NCORES) + cc
        lo = chunk * _CH
        zrows = _CHP // NSUB
        pltpu.sync_copy(zeros_hbm, acc_sh.at[pl.ds(s * zrows, zrows)])
        plsc.subcore_barrier()

        def step(r, carry):
            cnt, qidx = carry
            data = []
            for j in range(8):
                ids16 = ids_v[r, 0, pl.ds(j * 16, 16)]
                local = ids16 - lo
                m = local.astype(jnp.uint32) < jnp.uint32(_CH)
                csum = jnp.cumsum(m.astype(jnp.int32))
                data.append((local, m, csum, jnp.max(csum), j))
            run = cnt
            for local, m, csum, tot, j in data:
                pos = run - 1 + csum
                plsc.store_scatter(didx1, [pos], local, mask=m)
                w16 = wbase0 + r * _FQ + j * 16 + iota
                plsc.store_scatter(widx1, [pos], w16, mask=m)
                run = run + tot
            cnt = run

            def do_flush(carry2):
                cn, qi = carry2
                flush(qi)
                for j2 in range(8):
                    wtail = widx1[pl.ds(_SQ + j2 * 16, 16)]
                    dtail = didx1[pl.ds(_SQ + j2 * 16, 16)]
                    widx1[pl.ds(j2 * 16, 16)] = wtail
                    didx1[pl.ds(j2 * 16, 16)] = dtail
                return cn - _SQ, qi + 1

            return lax.cond(cnt >= _SQ, do_flush, lambda x: x, (cnt, qidx))

        cnt, qidx = lax.fori_loop(0, _IDROWS_PW, step,
                                  (jnp.int32(0), jnp.int32(0)))

        for j in range(_SQ // 16):
            pos = j * 16 + iota
            keep = pos < cnt
            dv = didx1[pl.ds(j * 16, 16)]
            wv = widx1[pl.ds(j * 16, 16)]
            didx1[pl.ds(j * 16, 16)] = jnp.where(keep, dv, _CH + iota)
            widx1[pl.ds(j * 16, 16)] = jnp.where(keep, wv, iota * 64)
        flush(qidx)
        wait_add()

        plsc.subcore_barrier()
        pltpu.sync_copy(acc_sh.at[pl.ds(s * 624, 624)],
                        seg_hbm.at[pl.ds(lo + s * 624, 624)])

        @pl.when(s == NSUB - 1)
        def _():
            pltpu.sync_copy(acc_sh.at[pl.ds(9360 + 624, 16)],
                            seg_hbm.at[pl.ds(lo + 9360 + 624, 16)])
        plsc.subcore_barrier()


def _scatter_call(bil, ids, zeros):
    mesh = plsc.VectorSubcoreMesh(core_axis_name="c", subcore_axis_name="s")
    f = pl.kernel(
        _scatter_body,
        out_type=jax.ShapeDtypeStruct((E, DP), jnp.float32),
        compiler_params=pltpu.CompilerParams(needs_layout_passes=False),
        mesh=mesh,
        scratch_types=[
            pltpu.VMEM_SHARED((_CHP, DP), jnp.float32),
            pltpu.VMEM((_IDROWS_PW, 1, _FQ), jnp.int32),
            pltpu.VMEM((2 * _SQ,), jnp.int32),
            pltpu.VMEM((2 * _SQ,), jnp.int32),
            pltpu.VMEM((1, _SQ), jnp.int32),
            pltpu.VMEM((1, _SQ), jnp.int32),
            pltpu.VMEM((_SQ, DP), jnp.float32),
            pltpu.SemaphoreType.DMA,
            pltpu.SemaphoreType.DMA,
        ],
    )
    return f(bil, ids, zeros)


def kernel(x, rbf, sbf, id_expand_kj, id_reduce_ji,
           W_rbf, W_sbf, W_ji, b_ji, W_kj, b_kj, W_bilin,
           Wb1, bb1, Wb2, bb2, W_fbs, b_fbs,
           Wa11, ba11, Wa12, ba12, Wa21, ba21, Wa22, ba22):
    x_ji, xk = _pre_call(x, rbf, W_rbf, W_ji, b_ji, W_kj, b_kj)

    pad = _GROWS * _GQ - T
    idp = jnp.concatenate(
        [id_expand_kj, (jnp.arange(pad, dtype=jnp.int32) * 37) % E])
    id2 = idp.reshape(_GROWS, 1, _GQ)
    xg3 = _gather_call(xk, id2)
    xg = xg3.reshape(_GROWS * _GQ, DP)

    Wb = W_bilin.reshape(D, D * D)
    bil = _bil_call(sbf, xg, W_sbf, Wb)

    idr = jnp.concatenate(
        [id_reduce_ji, jnp.full((pad,), E, jnp.int32)]).reshape(
            NSUB * _IDROWS_PW, 1, _FQ)
    zeros = jnp.zeros((_CHP // NSUB, DP), jnp.float32)
    seg = _scatter_call(bil, idr, zeros)

    return _post_call(x, x_ji, seg, Wb1, bb1, Wb2, bb2, W_fbs, b_fbs,
                      Wa11, ba11, Wa12, ba12, Wa21, ba21, Wa22, ba22)


# speedup vs baseline: 1.0988x; 1.0418x over previous
import functools

import jax
import jax.numpy as jnp
from jax import lax
from jax.experimental import pallas as pl
from jax.experimental.pallas import tpu as pltpu
from jax.experimental.pallas import tpu_sc as plsc

E = 160000
T = 160000
D = 64
DP = 128
NCORES = 2
NSUB = 16
NW = NCORES * NSUB


_BE = 2000


def _pre_body(x_ref, rbf_ref, wrbf_ref, wji_ref, bji_ref, wkj_ref, bkj_ref,
              xji_ref, xk_ref):
    xv = x_ref[...]
    g = jnp.dot(rbf_ref[...], wrbf_ref[...], preferred_element_type=jnp.float32)
    xji_ref[...] = jnp.dot(xv, wji_ref[...],
                           preferred_element_type=jnp.float32) + bji_ref[...]
    xk = (jnp.dot(xv, wkj_ref[...], preferred_element_type=jnp.float32)
          + bkj_ref[...]) * g
    xk_ref[...] = jnp.concatenate([xk, jnp.zeros_like(xk)], axis=1)


def _pre_call(x, rbf, W_rbf, W_ji, b_ji, W_kj, b_kj):
    n = x.shape[0] // _BE
    nr = rbf.shape[1]
    return pl.pallas_call(
        _pre_body,
        grid=(n,),
        in_specs=[
            pl.BlockSpec((_BE, D), lambda i: (i, 0)),
            pl.BlockSpec((_BE, nr), lambda i: (i, 0)),
            pl.BlockSpec((nr, D), lambda i: (0, 0)),
            pl.BlockSpec((D, D), lambda i: (0, 0)),
            pl.BlockSpec((1, D), lambda i: (0, 0)),
            pl.BlockSpec((D, D), lambda i: (0, 0)),
            pl.BlockSpec((1, D), lambda i: (0, 0)),
        ],
        out_specs=[
            pl.BlockSpec((_BE, D), lambda i: (i, 0)),
            pl.BlockSpec((_BE, DP), lambda i: (i, 0)),
        ],
        out_shape=[
            jax.ShapeDtypeStruct((x.shape[0], D), jnp.float32),
            jax.ShapeDtypeStruct((x.shape[0], DP), jnp.float32),
        ],
    )(x, rbf, W_rbf, W_ji, b_ji.reshape(1, D), W_kj, b_kj.reshape(1, D))


_BT = 3200


def _bil_body(sbf_ref, xg_ref, wsbf_ref, wb_ref, out_ref):
    bt = sbf_ref.shape[0]
    sbf_e = jnp.dot(sbf_ref[...], wsbf_ref[...],
                    preferred_element_type=jnp.float32)
    sbf_et = sbf_e.astype(jnp.bfloat16).T
    xgt = xg_ref[...][:, :D].astype(jnp.bfloat16).T
    mt = (sbf_et[:, None, :] * xgt[None, :, :]).reshape(D * D, bt)
    res = jnp.dot(wb_ref[...], mt, preferred_element_type=jnp.float32).T
    out_ref[...] = jnp.concatenate([res, jnp.zeros_like(res)], axis=1)


def _bil_call(sbf, xg, W_sbf, Wb):
    n = sbf.shape[0] // _BT
    ns = sbf.shape[1]
    return pl.pallas_call(
        _bil_body,
        grid=(n,),
        in_specs=[
            pl.BlockSpec((_BT, ns), lambda i: (i, 0)),
            pl.BlockSpec((_BT, DP), lambda i: (i, 0)),
            pl.BlockSpec((ns, D), lambda i: (0, 0)),
            pl.BlockSpec((D, D * D), lambda i: (0, 0)),
        ],
        out_specs=pl.BlockSpec((_BT, DP), lambda i: (i, 0)),
        out_shape=jax.ShapeDtypeStruct((sbf.shape[0], DP), jnp.float32),
    )(sbf, xg, W_sbf, Wb.astype(jnp.bfloat16))


def _silu(v):
    return v * jax.nn.sigmoid(v)


def _post_body(x_ref, xji_ref, seg_ref,
               wb1_ref, bb1_ref, wb2_ref, bb2_ref, wfbs_ref, bfbs_ref,
               wa11_ref, ba11_ref, wa12_ref, ba12_ref,
               wa21_ref, ba21_ref, wa22_ref, ba22_ref, out_ref):
    def dot(a, w_ref, b_ref):
        return jnp.dot(a.astype(jnp.bfloat16), w_ref[...],
                       preferred_element_type=jnp.float32) + b_ref[...]

    h = xji_ref[...] + seg_ref[...][:, :D]
    h = h + dot(_silu(dot(h, wb1_ref, bb1_ref)), wb2_ref, bb2_ref)
    h = _silu(h)
    h = _silu(dot(h, wfbs_ref, bfbs_ref))
    xo = x_ref[...] + h
    xo = xo + dot(_silu(dot(xo, wa11_ref, ba11_ref)), wa12_ref, ba12_ref)
    xo = _silu(xo)
    xo = xo + dot(_silu(dot(xo, wa21_ref, ba21_ref)), wa22_ref, ba22_ref)
    out_ref[...] = _silu(xo)


def _post_call(x, x_ji, seg, Wb1, bb1, Wb2, bb2, W_fbs, b_fbs,
               Wa11, ba11, Wa12, ba12, Wa21, ba21, Wa22, ba22):
    n = x.shape[0] // _BE
    mat = pl.BlockSpec((D, D), lambda i: (0, 0))
    vec = pl.BlockSpec((1, D), lambda i: (0, 0))
    big = pl.BlockSpec((_BE, D), lambda i: (i, 0))
    segspec = pl.BlockSpec((_BE, DP), lambda i: (i, 0))
    return pl.pallas_call(
        _post_body,
        grid=(n,),
        in_specs=[big, big, segspec,
                  mat, vec, mat, vec, mat, vec,
                  mat, vec, mat, vec, mat, vec, mat, vec],
        out_specs=big,
        out_shape=jax.ShapeDtypeStruct((x.shape[0], D), jnp.float32),
    )(x, x_ji, seg,
      Wb1.astype(jnp.bfloat16), bb1.reshape(1, D),
      Wb2.astype(jnp.bfloat16), bb2.reshape(1, D),
      W_fbs.astype(jnp.bfloat16), b_fbs.reshape(1, D),
      Wa11.astype(jnp.bfloat16), ba11.reshape(1, D),
      Wa12.astype(jnp.bfloat16), ba12.reshape(1, D),
      Wa21.astype(jnp.bfloat16), ba21.reshape(1, D),
      Wa22.astype(jnp.bfloat16), ba22.reshape(1, D))


_GQ = 128
_GROWS = 1280
_GPW = _GROWS // NW


def _gather_body(xk_hbm, id3_hbm, out_hbm, idx_v, rows_v, isem, gsem, osem):
    w = lax.axis_index("s") * NCORES + lax.axis_index("c")
    nslot = 3

    store_started, store_waited = [], []

    def wait_store(j):
        pltpu.make_async_copy(rows_v.at[j % nslot],
                              out_hbm.at[w * _GPW + j], osem).wait()
        store_waited.append(j)

    for k in range(_GPW + 2):
        if k >= 2:
            km2 = k - 2
            pltpu.make_async_copy(
                xk_hbm.at[idx_v.at[km2 % nslot, 0]], rows_v.at[km2 % nslot],
                gsem).wait()
            pltpu.async_copy(rows_v.at[km2 % nslot],
                             out_hbm.at[w * _GPW + km2], osem)
            store_started.append(km2)
        if k < _GPW:
            pltpu.async_copy(id3_hbm.at[w * _GPW + k], idx_v.at[k % nslot],
                             isem)
        if 1 <= k <= _GPW:
            km1 = k - 1
            if km1 - nslot >= 0 and (km1 - nslot) not in store_waited:
                wait_store(km1 - nslot)
            pltpu.make_async_copy(id3_hbm.at[w * _GPW + km1],
                                  idx_v.at[km1 % nslot], isem).wait()
            pltpu.async_copy(
                xk_hbm.at[idx_v.at[km1 % nslot, 0]], rows_v.at[km1 % nslot],
                gsem)
    for j in store_started:
        if j not in store_waited:
            wait_store(j)


def _gather_call(xk, id2):
    mesh = plsc.VectorSubcoreMesh(core_axis_name="c", subcore_axis_name="s")
    f = pl.kernel(
        _gather_body,
        out_type=jax.ShapeDtypeStruct((_GROWS, _GQ, DP), jnp.float32),
        mesh=mesh,
        scratch_types=[
            pltpu.VMEM((3, 1, _GQ), jnp.int32),
            pltpu.VMEM((3, _GQ, DP), jnp.float32),
            pltpu.SemaphoreType.DMA,
            pltpu.SemaphoreType.DMA,
            pltpu.SemaphoreType.DMA,
        ],
    )
    return f(xk, id2)


_NCHUNK = 16
_CH = E // _NCHUNK
_CHP = 10240
_IDROWS_PW = 80
_FQ = 128
_SQ = 128


def _scatter_body(bil_hbm, ids_hbm, zeros_hbm, seg_hbm,
                  acc_sh, ids_v, widx1, didx1, widx2, didx2, rows_v,
                  ssem, asem):
    c = lax.axis_index("c")
    s = lax.axis_index("s")
    iota = lax.iota(jnp.int32, 16)

    pltpu.sync_copy(ids_hbm.at[pl.ds(s * _IDROWS_PW, _IDROWS_PW)], ids_v)
    wbase0 = s * (_IDROWS_PW * _FQ)

    def wait_add():
        pltpu.make_async_copy(bil_hbm.at[pl.ds(0, _SQ)], rows_v, asem).wait()

    def flush(qidx):
        lax.cond(qidx > 0, wait_add, lambda: None)
        for j in range(_SQ // 16):
            widx2[0, pl.ds(j * 16, 16)] = widx1[pl.ds(j * 16, 16)]
            didx2[0, pl.ds(j * 16, 16)] = didx1[pl.ds(j * 16, 16)]
        pltpu.async_copy(bil_hbm.at[widx2.at[0]], rows_v, ssem).wait()
        pltpu.async_copy(rows_v, acc_sh.at[didx2.at[0]], asem, add=True)

    for cc in range(_NCHUNK // NCORES):
        chunk = c * (_NCHUNK // NCORES) + cc
        lo = chunk * _CH
        zrows = _CHP // NSUB
        pltpu.sync_copy(zeros_hbm, acc_sh.at[pl.ds(s * zrows, zrows)])
        plsc.subcore_barrier()

        def step(r, carry):
            cnt, qidx = carry
            data = []
            for j in range(8):
                ids16 = ids_v[r, 0, pl.ds(j * 16, 16)]
                local = ids16 - lo
                m = local.astype(jnp.uint32) < jnp.uint32(_CH)
                csum = jnp.cumsum(m.astype(jnp.int32))
                data.append((local, m, csum, jnp.max(csum), j))
            run = cnt
            for local, m, csum, tot, j in data:
                pos = run - 1 + csum
                plsc.store_scatter(didx1, [pos], local, mask=m)
                w16 = wbase0 + r * _FQ + j * 16 + iota
                plsc.store_scatter(widx1, [pos], w16, mask=m)
                run = run + tot
            cnt = run

            def do_flush(carry2):
                cn, qi = carry2
                flush(qi)
                for j2 in range(8):
                    wtail = widx1[pl.ds(_SQ + j2 * 16, 16)]
                    dtail = didx1[pl.ds(_SQ + j2 * 16, 16)]
                    widx1[pl.ds(j2 * 16, 16)] = wtail
                    didx1[pl.ds(j2 * 16, 16)] = dtail
                return cn - _SQ, qi + 1

            return lax.cond(cnt >= _SQ, do_flush, lambda x: x, (cnt, qidx))

        cnt, qidx = lax.fori_loop(0, _IDROWS_PW, step,
                                  (jnp.int32(0), jnp.int32(0)))

        for j in range(_SQ // 16):
            pos = j * 16 + iota
            keep = pos < cnt
            dv = didx1[pl.ds(j * 16, 16)]
            wv = widx1[pl.ds(j * 16, 16)]
            didx1[pl.ds(j * 16, 16)] = jnp.where(keep, dv, _CH + iota)
            widx1[pl.ds(j * 16, 16)] = jnp.where(keep, wv, iota * 64)
        flush(qidx)
        wait_add()

        plsc.subcore_barrier()
        pltpu.sync_copy(acc_sh.at[pl.ds(s * 624, 624)],
                        seg_hbm.at[pl.ds(lo + s * 624, 624)])

        @pl.when(s == NSUB - 1)
        def _():
            pltpu.sync_copy(acc_sh.at[pl.ds(9360 + 624, 16)],
                            seg_hbm.at[pl.ds(lo + 9360 + 624, 16)])
        plsc.subcore_barrier()


def _scatter_call(bil, ids, zeros):
    mesh = plsc.VectorSubcoreMesh(core_axis_name="c", subcore_axis_name="s")
    f = pl.kernel(
        _scatter_body,
        out_type=jax.ShapeDtypeStruct((E, DP), jnp.float32),
        compiler_params=pltpu.CompilerParams(needs_layout_passes=False),
        mesh=mesh,
        scratch_types=[
            pltpu.VMEM_SHARED((_CHP, DP), jnp.float32),
            pltpu.VMEM((_IDROWS_PW, 1, _FQ), jnp.int32),
            pltpu.VMEM((2 * _SQ,), jnp.int32),
            pltpu.VMEM((2 * _SQ,), jnp.int32),
            pltpu.VMEM((1, _SQ), jnp.int32),
            pltpu.VMEM((1, _SQ), jnp.int32),
            pltpu.VMEM((_SQ, DP), jnp.float32),
            pltpu.SemaphoreType.DMA,
            pltpu.SemaphoreType.DMA,
        ],
    )
    return f(bil, ids, zeros)


def kernel(x, rbf, sbf, id_expand_kj, id_reduce_ji,
           W_rbf, W_sbf, W_ji, b_ji, W_kj, b_kj, W_bilin,
           Wb1, bb1, Wb2, bb2, W_fbs, b_fbs,
           Wa11, ba11, Wa12, ba12, Wa21, ba21, Wa22, ba22):
    x_ji, xk = _pre_call(x, rbf, W_rbf, W_ji, b_ji, W_kj, b_kj)

    pad = _GROWS * _GQ - T
    idp = jnp.concatenate(
        [id_expand_kj, (jnp.arange(pad, dtype=jnp.int32) * 37) % E])
    id2 = idp.reshape(_GROWS, 1, _GQ)
    xg3 = _gather_call(xk, id2)
    xg = xg3.reshape(_GROWS * _GQ, DP)

    Wb = W_bilin.reshape(D, D * D)
    bil = _bil_call(sbf, xg, W_sbf, Wb)

    idr = jnp.concatenate(
        [id_reduce_ji, jnp.full((pad,), E, jnp.int32)]).reshape(
            NSUB * _IDROWS_PW, 1, _FQ)
    zeros = jnp.zeros((_CHP // NSUB, DP), jnp.float32)
    seg = _scatter_call(bil, idr, zeros)

    return _post_call(x, x_ji, seg, Wb1, bb1, Wb2, bb2, W_fbs, b_fbs,
                      Wa11, ba11, Wa12, ba12, Wa21, ba21, Wa22, ba22)


# speedup vs baseline: 1.1009x; 1.0020x over previous
import jax
import jax.numpy as jnp
from jax import lax
from jax.experimental import pallas as pl
from jax.experimental.pallas import tpu as pltpu
from jax.experimental.pallas import tpu_sc as plsc

E = 160000
T = 160000
D = 64
DP = 128
NCORES = 2
NSUB = 16
NW = NCORES * NSUB


_BE = 2000


def _pre_body(x_ref, rbf_ref, wrbf_ref, wji_ref, bji_ref, wkj_ref, bkj_ref,
              xji_ref, xk_ref):
    xv = x_ref[...]
    g = jnp.dot(rbf_ref[...], wrbf_ref[...], preferred_element_type=jnp.float32)
    xji_ref[...] = jnp.dot(xv, wji_ref[...],
                           preferred_element_type=jnp.float32) + bji_ref[...]
    xk = (jnp.dot(xv, wkj_ref[...], preferred_element_type=jnp.float32)
          + bkj_ref[...]) * g
    xk_ref[...] = jnp.concatenate([xk, jnp.zeros_like(xk)], axis=1)


def _pre_call(x, rbf, W_rbf, W_ji, b_ji, W_kj, b_kj):
    n = x.shape[0] // _BE
    nr = rbf.shape[1]
    return pl.pallas_call(
        _pre_body,
        grid=(n,),
        in_specs=[
            pl.BlockSpec((_BE, D), lambda i: (i, 0)),
            pl.BlockSpec((_BE, nr), lambda i: (i, 0)),
            pl.BlockSpec((nr, D), lambda i: (0, 0)),
            pl.BlockSpec((D, D), lambda i: (0, 0)),
            pl.BlockSpec((1, D), lambda i: (0, 0)),
            pl.BlockSpec((D, D), lambda i: (0, 0)),
            pl.BlockSpec((1, D), lambda i: (0, 0)),
        ],
        out_specs=[
            pl.BlockSpec((_BE, D), lambda i: (i, 0)),
            pl.BlockSpec((_BE, DP), lambda i: (i, 0)),
        ],
        out_shape=[
            jax.ShapeDtypeStruct((x.shape[0], D), jnp.float32),
            jax.ShapeDtypeStruct((x.shape[0], DP), jnp.float32),
        ],
    )(x, rbf, W_rbf, W_ji, b_ji.reshape(1, D), W_kj, b_kj.reshape(1, D))


_BT = 3200


def _bil_body(sbf_ref, xg_ref, wsbf_ref, wb_ref, out_ref):
    bt = sbf_ref.shape[0]
    sbf_e = jnp.dot(sbf_ref[...], wsbf_ref[...],
                    preferred_element_type=jnp.float32)
    sbf_et = sbf_e.astype(jnp.bfloat16).T
    xgt = xg_ref[...][:, :D].astype(jnp.bfloat16).T
    mt = (sbf_et[:, None, :] * xgt[None, :, :]).reshape(D * D, bt)
    res = jnp.dot(wb_ref[...], mt, preferred_element_type=jnp.float32).T
    out_ref[...] = jnp.concatenate([res, jnp.zeros_like(res)], axis=1)


def _bil_call(sbf, xg, W_sbf, Wb):
    n = sbf.shape[0] // _BT
    ns = sbf.shape[1]
    return pl.pallas_call(
        _bil_body,
        grid=(n,),
        in_specs=[
            pl.BlockSpec((_BT, ns), lambda i: (i, 0)),
            pl.BlockSpec((_BT, DP), lambda i: (i, 0)),
            pl.BlockSpec((ns, D), lambda i: (0, 0)),
            pl.BlockSpec((D, D * D), lambda i: (0, 0)),
        ],
        out_specs=pl.BlockSpec((_BT, DP), lambda i: (i, 0)),
        out_shape=jax.ShapeDtypeStruct((sbf.shape[0], DP), jnp.float32),
    )(sbf, xg, W_sbf, Wb.astype(jnp.bfloat16))


def _silu(v):
    return v * jax.nn.sigmoid(v)


def _post_body(x_ref, xji_ref, seg_ref,
               wb1_ref, bb1_ref, wb2_ref, bb2_ref, wfbs_ref, bfbs_ref,
               wa11_ref, ba11_ref, wa12_ref, ba12_ref,
               wa21_ref, ba21_ref, wa22_ref, ba22_ref, out_ref):
    def dot(a, w_ref, b_ref):
        return jnp.dot(a.astype(jnp.bfloat16), w_ref[...],
                       preferred_element_type=jnp.float32) + b_ref[...]

    h = xji_ref[...] + seg_ref[...][:, :D]
    h = h + dot(_silu(dot(h, wb1_ref, bb1_ref)), wb2_ref, bb2_ref)
    h = _silu(h)
    h = _silu(dot(h, wfbs_ref, bfbs_ref))
    xo = x_ref[...] + h
    xo = xo + dot(_silu(dot(xo, wa11_ref, ba11_ref)), wa12_ref, ba12_ref)
    xo = _silu(xo)
    xo = xo + dot(_silu(dot(xo, wa21_ref, ba21_ref)), wa22_ref, ba22_ref)
    out_ref[...] = _silu(xo)


def _post_call(x, x_ji, seg, Wb1, bb1, Wb2, bb2, W_fbs, b_fbs,
               Wa11, ba11, Wa12, ba12, Wa21, ba21, Wa22, ba22):
    n = x.shape[0] // _BE
    mat = pl.BlockSpec((D, D), lambda i: (0, 0))
    vec = pl.BlockSpec((1, D), lambda i: (0, 0))
    big = pl.BlockSpec((_BE, D), lambda i: (i, 0))
    segspec = pl.BlockSpec((_BE, DP), lambda i: (i, 0))
    return pl.pallas_call(
        _post_body,
        grid=(n,),
        in_specs=[big, big, segspec,
                  mat, vec, mat, vec, mat, vec,
                  mat, vec, mat, vec, mat, vec, mat, vec],
        out_specs=big,
        out_shape=jax.ShapeDtypeStruct((x.shape[0], D), jnp.float32),
    )(x, x_ji, seg,
      Wb1.astype(jnp.bfloat16), bb1.reshape(1, D),
      Wb2.astype(jnp.bfloat16), bb2.reshape(1, D),
      W_fbs.astype(jnp.bfloat16), b_fbs.reshape(1, D),
      Wa11.astype(jnp.bfloat16), ba11.reshape(1, D),
      Wa12.astype(jnp.bfloat16), ba12.reshape(1, D),
      Wa21.astype(jnp.bfloat16), ba21.reshape(1, D),
      Wa22.astype(jnp.bfloat16), ba22.reshape(1, D))


_GQ = 128
_GROWS = 1280
_GPW = _GROWS // NW


def _gather_body(xk_hbm, id3_hbm, out_hbm, idx_v, rows_v, isem, gsem, osem):
    w = lax.axis_index("s") * NCORES + lax.axis_index("c")
    nslot = 3

    store_started, store_waited = [], []

    def wait_store(j):
        pltpu.make_async_copy(rows_v.at[j % nslot],
                              out_hbm.at[w * _GPW + j], osem).wait()
        store_waited.append(j)

    for k in range(_GPW + 2):
        if k >= 2:
            km2 = k - 2
            pltpu.make_async_copy(
                xk_hbm.at[idx_v.at[km2 % nslot, 0]], rows_v.at[km2 % nslot],
                gsem).wait()
            pltpu.async_copy(rows_v.at[km2 % nslot],
                             out_hbm.at[w * _GPW + km2], osem)
            store_started.append(km2)
        if k < _GPW:
            pltpu.async_copy(id3_hbm.at[w * _GPW + k], idx_v.at[k % nslot],
                             isem)
        if 1 <= k <= _GPW:
            km1 = k - 1
            if km1 - nslot >= 0 and (km1 - nslot) not in store_waited:
                wait_store(km1 - nslot)
            pltpu.make_async_copy(id3_hbm.at[w * _GPW + km1],
                                  idx_v.at[km1 % nslot], isem).wait()
            pltpu.async_copy(
                xk_hbm.at[idx_v.at[km1 % nslot, 0]], rows_v.at[km1 % nslot],
                gsem)
    for j in store_started:
        if j not in store_waited:
            wait_store(j)


def _gather_call(xk, id2):
    mesh = plsc.VectorSubcoreMesh(core_axis_name="c", subcore_axis_name="s")
    f = pl.kernel(
        _gather_body,
        out_type=jax.ShapeDtypeStruct((_GROWS, _GQ, DP), jnp.float32),
        mesh=mesh,
        scratch_types=[
            pltpu.VMEM((3, 1, _GQ), jnp.int32),
            pltpu.VMEM((3, _GQ, DP), jnp.float32),
            pltpu.SemaphoreType.DMA,
            pltpu.SemaphoreType.DMA,
            pltpu.SemaphoreType.DMA,
        ],
    )
    return f(xk, id2)


_NCHUNK = 16
_CH = E // _NCHUNK
_CHP = 10240
_IDROWS_PW = 80
_FQ = 128
_SQ = 128


def _scatter_body(bil_hbm, ids_hbm, zeros_hbm, seg_hbm,
                  acc_sh, ids_v, widx1, didx1, widx2, didx2, rows_v,
                  ssem, asem):
    c = lax.axis_index("c")
    s = lax.axis_index("s")
    iota = lax.iota(jnp.int32, 16)

    pltpu.sync_copy(ids_hbm.at[pl.ds(s * _IDROWS_PW, _IDROWS_PW)], ids_v)
    wbase0 = s * (_IDROWS_PW * _FQ)

    def wait_add():
        pltpu.make_async_copy(bil_hbm.at[pl.ds(0, _SQ)], rows_v, asem).wait()

    def flush(qidx):
        lax.cond(qidx > 0, wait_add, lambda: None)
        for j in range(_SQ // 16):
            widx2[0, pl.ds(j * 16, 16)] = widx1[pl.ds(j * 16, 16)]
            didx2[0, pl.ds(j * 16, 16)] = didx1[pl.ds(j * 16, 16)]
        pltpu.async_copy(bil_hbm.at[widx2.at[0]], rows_v, ssem).wait()
        pltpu.async_copy(rows_v, acc_sh.at[didx2.at[0]], asem, add=True)

    for cc in range(_NCHUNK // NCORES):
        chunk = c * (_NCHUNK // NCORES) + cc
        lo = chunk * _CH
        zrows = _CHP // NSUB
        pltpu.sync_copy(zeros_hbm, acc_sh.at[pl.ds(s * zrows, zrows)])
        plsc.subcore_barrier()

        def step(r, carry):
            cnt, qidx = carry
            data = []
            for j in range(8):
                ids16 = ids_v[r, 0, pl.ds(j * 16, 16)]
                local = ids16 - lo
                m = local.astype(jnp.uint32) < jnp.uint32(_CH)
                csum = jnp.cumsum(m.astype(jnp.int32))
                data.append((local, m, csum, jnp.max(csum), j))
            run = cnt
            for local, m, csum, tot, j in data:
                pos = run - 1 + csum
                plsc.store_scatter(didx1, [pos], local, mask=m)
                w16 = wbase0 + r * _FQ + j * 16 + iota
                plsc.store_scatter(widx1, [pos], w16, mask=m)
                run = run + tot
            cnt = run

            def do_flush(carry2):
                cn, qi = carry2
                flush(qi)
                for j2 in range(8):
                    wtail = widx1[pl.ds(_SQ + j2 * 16, 16)]
                    dtail = didx1[pl.ds(_SQ + j2 * 16, 16)]
                    widx1[pl.ds(j2 * 16, 16)] = wtail
                    didx1[pl.ds(j2 * 16, 16)] = dtail
                return cn - _SQ, qi + 1

            return lax.cond(cnt >= _SQ, do_flush, lambda x: x, (cnt, qidx))

        cnt, qidx = lax.fori_loop(0, _IDROWS_PW, step,
                                  (jnp.int32(0), jnp.int32(0)))

        for j in range(_SQ // 16):
            pos = j * 16 + iota
            keep = pos < cnt
            dv = didx1[pl.ds(j * 16, 16)]
            wv = widx1[pl.ds(j * 16, 16)]
            didx1[pl.ds(j * 16, 16)] = jnp.where(keep, dv, _CH + iota)
            widx1[pl.ds(j * 16, 16)] = jnp.where(keep, wv, iota * 64)
        flush(qidx)
        wait_add()

        plsc.subcore_barrier()
        pltpu.sync_copy(acc_sh.at[pl.ds(s * 624, 624)],
                        seg_hbm.at[pl.ds(lo + s * 624, 624)])

        @pl.when(s == NSUB - 1)
        def _():
            pltpu.sync_copy(acc_sh.at[pl.ds(9360 + 624, 16)],
                            seg_hbm.at[pl.ds(lo + 9360 + 624, 16)])
        plsc.subcore_barrier()


def _scatter_call(bil, ids, zeros):
    mesh = plsc.VectorSubcoreMesh(core_axis_name="c", subcore_axis_name="s")
    f = pl.kernel(
        _scatter_body,
        out_type=jax.ShapeDtypeStruct((E, DP), jnp.float32),
        compiler_params=pltpu.CompilerParams(needs_layout_passes=False),
        mesh=mesh,
        scratch_types=[
            pltpu.VMEM_SHARED((_CHP, DP), jnp.float32),
            pltpu.VMEM((_IDROWS_PW, 1, _FQ), jnp.int32),
            pltpu.VMEM((2 * _SQ,), jnp.int32),
            pltpu.VMEM((2 * _SQ,), jnp.int32),
            pltpu.VMEM((1, _SQ), jnp.int32),
            pltpu.VMEM((1, _SQ), jnp.int32),
            pltpu.VMEM((_SQ, DP), jnp.float32),
            pltpu.SemaphoreType.DMA,
            pltpu.SemaphoreType.DMA,
        ],
    )
    return f(bil, ids, zeros)


def kernel(x, rbf, sbf, id_expand_kj, id_reduce_ji,
           W_rbf, W_sbf, W_ji, b_ji, W_kj, b_kj, W_bilin,
           Wb1, bb1, Wb2, bb2, W_fbs, b_fbs,
           Wa11, ba11, Wa12, ba12, Wa21, ba21, Wa22, ba22):
    x_ji, xk = _pre_call(x, rbf, W_rbf, W_ji, b_ji, W_kj, b_kj)

    pad = _GROWS * _GQ - T
    idp = jnp.concatenate(
        [id_expand_kj, (jnp.arange(pad, dtype=jnp.int32) * 37) % E])
    id2 = idp.reshape(_GROWS, 1, _GQ)
    xg3 = _gather_call(xk, id2)
    xg = xg3.reshape(_GROWS * _GQ, DP)

    Wb = W_bilin.reshape(D, D * D)
    bil = _bil_call(sbf, xg, W_sbf, Wb)

    idr = jnp.concatenate(
        [id_reduce_ji, jnp.full((pad,), E, jnp.int32)]).reshape(
            NSUB * _IDROWS_PW, 1, _FQ)
    zeros = jnp.zeros((_CHP // NSUB, DP), jnp.float32)
    seg = _scatter_call(bil, idr, zeros)

    return _post_call(x, x_ji, seg, Wb1, bb1, Wb2, bb2, W_fbs, b_fbs,
                      Wa11, ba11, Wa12, ba12, Wa21, ba21, Wa22, ba22)
